# baseline (speedup 1.0000x reference)
import jax
import jax.numpy as jnp
from jax.experimental import pallas as pl
from jax.experimental.pallas import tpu as pltpu


def _vq_body(h_ref, w_ref, q_ref, loss_ref):
    i = pl.program_id(0)
    hb = h_ref[...]
    w = w_ref[...]
    h_sq = jnp.sum(hb * hb, axis=1, keepdims=True)
    w_sq = jnp.sum(w * w, axis=1)
    m = jax.lax.dot_general(hb, w, (((1,), (1,)), ((), ())),
                            preferred_element_type=jnp.float32)
    dist = (h_sq + w_sq[None, :]) - 2.0 * m
    minval = jnp.min(dist, axis=1, keepdims=True)
    K = w.shape[0]
    iota = jax.lax.broadcasted_iota(jnp.int32, dist.shape, 1)
    idx = jnp.min(jnp.where(dist == minval, iota, K), axis=1)
    onehot = (iota == idx[:, None]).astype(jnp.float32)
    q_ref[...] = jax.lax.dot_general(
        onehot, w, (((1,), (0,)), ((), ())),
        preferred_element_type=jnp.float32,
        precision=jax.lax.Precision.HIGHEST)

    @pl.when(i == 0)
    def _():
        loss_ref[0, 0] = 0.0

    loss_ref[0, 0] += jnp.sum(minval)


def kernel(h, W):
    N = h.shape[0] * h.shape[1]
    D = h.shape[2]
    K = W.shape[0]
    h_flat = h.reshape(N, D)
    BN = 512
    grid = N // BN

    q, loss_sum = pl.pallas_call(
        _vq_body,
        grid=(grid,),
        in_specs=[
            pl.BlockSpec((BN, D), lambda i: (i, 0)),
            pl.BlockSpec((K, D), lambda i: (0, 0)),
        ],
        out_specs=[
            pl.BlockSpec((BN, D), lambda i: (i, 0)),
            pl.BlockSpec((1, 1), lambda i: (0, 0), memory_space=pltpu.SMEM),
        ],
        out_shape=[
            jax.ShapeDtypeStruct((N, D), jnp.float32),
            jax.ShapeDtypeStruct((1, 1), jnp.float32),
        ],
        compiler_params=pltpu.CompilerParams(
            dimension_semantics=("arbitrary",)),
    )(h_flat, W)

    mse = loss_sum[0, 0] / jnp.float32(N * D)
    commitment_loss = jnp.float32(0.25) * mse
    codebook_loss = mse
    return q.reshape(h.shape), commitment_loss, codebook_loss


# speedup vs baseline: 1.4044x; 1.4044x over previous
import functools

import jax
import jax.numpy as jnp
from jax import lax
from jax.experimental import pallas as pl
from jax.experimental.pallas import tpu as pltpu
from jax.experimental.pallas import tpu_sc as plsc

_NC = 2
_NS = 16
_NW = _NC * _NS
_CHUNK = 128


def _vq_tc_body(h_ref, w_ref, idx_ref, loss_ref):
    i = pl.program_id(0)
    hb = h_ref[...]
    w = w_ref[...]
    h_sq = jnp.sum(hb * hb, axis=1, keepdims=True)
    w_sq = jnp.sum(w * w, axis=1)
    m = jax.lax.dot_general(hb, w, (((1,), (1,)), ((), ())),
                            preferred_element_type=jnp.float32)
    dist = (h_sq + w_sq[None, :]) - 2.0 * m
    minval = jnp.min(dist, axis=1, keepdims=True)
    K = w.shape[0]
    iota = jax.lax.broadcasted_iota(jnp.int32, dist.shape, 1)
    idx = jnp.min(jnp.where(dist == minval, iota, K), axis=1)
    idx_ref[0, 0, :] = idx

    @pl.when(i == 0)
    def _():
        loss_ref[0, 0] = 0.0

    loss_ref[0, 0] += jnp.sum(minval)


def _tc_stage(h_flat, W, BN):
    N, D = h_flat.shape
    K = W.shape[0]
    grid = N // BN
    idx3, loss_sum = pl.pallas_call(
        _vq_tc_body,
        grid=(grid,),
        in_specs=[
            pl.BlockSpec((BN, D), lambda i: (i, 0)),
            pl.BlockSpec((K, D), lambda i: (0, 0)),
        ],
        out_specs=[
            pl.BlockSpec((1, 1, BN), lambda i: (i, 0, 0)),
            pl.BlockSpec((1, 1), lambda i: (0, 0), memory_space=pltpu.SMEM),
        ],
        out_shape=[
            jax.ShapeDtypeStruct((grid, 1, BN), jnp.int32),
            jax.ShapeDtypeStruct((1, 1), jnp.float32),
        ],
        compiler_params=pltpu.CompilerParams(
            dimension_semantics=("arbitrary",)),
    )(h_flat, W)
    return idx3.reshape(N), loss_sum


def _make_sc_gather(N, K, D):
    b_per_w = N // _NW
    n_chunks = b_per_w // _CHUNK
    mesh = plsc.VectorSubcoreMesh(core_axis_name="c", subcore_axis_name="s")

    @functools.partial(
        pl.kernel,
        mesh=mesh,
        out_type=jax.ShapeDtypeStruct((N, D), jnp.float32),
        compiler_params=pltpu.CompilerParams(use_tc_tiling_on_sc=False),
        scratch_types=[
            pltpu.VMEM((b_per_w,), jnp.int32),
            pltpu.VMEM((b_per_w, D), jnp.float32),
            pltpu.SemaphoreType.DMA,
        ],
    )
    def gather_kernel(idx_hbm, table_hbm, out_hbm, idx_v, rows_v, sem):
        wid = lax.axis_index("s") * _NC + lax.axis_index("c")
        base = wid * b_per_w
        pltpu.sync_copy(idx_hbm.at[pl.ds(base, b_per_w)], idx_v)
        copies = []
        for c in range(n_chunks):
            copies.append(pltpu.async_copy(
                table_hbm.at[idx_v.at[pl.ds(c * _CHUNK, _CHUNK)]],
                rows_v.at[pl.ds(c * _CHUNK, _CHUNK)],
                sem))
        for cp in copies:
            cp.wait()
        pltpu.sync_copy(rows_v, out_hbm.at[pl.ds(base, b_per_w)])

    return gather_kernel


def kernel(h, W):
    N = h.shape[0] * h.shape[1]
    D = h.shape[2]
    K = W.shape[0]
    h_flat = h.reshape(N, D)

    idx, loss_sum = _tc_stage(h_flat, W, BN=512)
    q = _make_sc_gather(N, K, D)(idx, W)

    mse = loss_sum[0, 0] / jnp.float32(N * D)
    commitment_loss = jnp.float32(0.25) * mse
    codebook_loss = mse
    return q.reshape(h.shape), commitment_loss, codebook_loss


# speedup vs baseline: 1.7529x; 1.2482x over previous
import functools

import jax
import jax.numpy as jnp
from jax import lax
from jax.experimental import pallas as pl
from jax.experimental.pallas import tpu as pltpu
from jax.experimental.pallas import tpu_sc as plsc

_NC = 2
_NS = 16
_NW = _NC * _NS
_CHUNK = 128


def _vq_tc_body(h_ref, w_ref, idx_ref, loss_ref):
    i = pl.program_id(0)
    hb = h_ref[...]
    w = w_ref[...]
    K = w.shape[0]
    w_sq = jnp.sum(w * w, axis=1, keepdims=True)
    h_sq = jax.lax.dot_general(
        jnp.ones((1, hb.shape[1]), jnp.float32), hb * hb,
        (((1,), (1,)), ((), ())),
        preferred_element_type=jnp.float32)
    m2 = jax.lax.dot_general(-2.0 * w, hb, (((1,), (1,)), ((), ())),
                             preferred_element_type=jnp.float32)
    dist = (h_sq + w_sq) + m2
    minval = jnp.min(dist, axis=0, keepdims=True)
    iota_f = jax.lax.broadcasted_iota(
        jnp.int32, dist.shape, 0).astype(jnp.float32)
    idx_f = jnp.min(jnp.where(dist == minval, iota_f, float(K)), axis=0)
    idx_ref[0, 0, :] = idx_f.astype(jnp.int32)

    @pl.when(i == 0)
    def _():
        loss_ref[0, 0] = 0.0

    loss_ref[0, 0] += jnp.sum(minval)


def _tc_stage(h_flat, W, BN):
    N, D = h_flat.shape
    K = W.shape[0]
    grid = N // BN
    idx3, loss_sum = pl.pallas_call(
        _vq_tc_body,
        grid=(grid,),
        in_specs=[
            pl.BlockSpec((BN, D), lambda i: (i, 0)),
            pl.BlockSpec((K, D), lambda i: (0, 0)),
        ],
        out_specs=[
            pl.BlockSpec((1, 1, BN), lambda i: (i, 0, 0)),
            pl.BlockSpec((1, 1), lambda i: (0, 0), memory_space=pltpu.SMEM),
        ],
        out_shape=[
            jax.ShapeDtypeStruct((grid, 1, BN), jnp.int32),
            jax.ShapeDtypeStruct((1, 1), jnp.float32),
        ],
        compiler_params=pltpu.CompilerParams(
            dimension_semantics=("arbitrary",)),
    )(h_flat, W)
    return idx3.reshape(N), loss_sum


def _make_sc_gather(N, K, D):
    b_per_w = N // _NW
    n_chunks = b_per_w // _CHUNK
    mesh = plsc.VectorSubcoreMesh(core_axis_name="c", subcore_axis_name="s")

    @functools.partial(
        pl.kernel,
        mesh=mesh,
        out_type=jax.ShapeDtypeStruct((N, D), jnp.float32),
        compiler_params=pltpu.CompilerParams(use_tc_tiling_on_sc=False),
        scratch_types=[
            pltpu.VMEM((b_per_w,), jnp.int32),
            pltpu.VMEM((b_per_w, D), jnp.float32),
            pltpu.SemaphoreType.DMA,
        ],
    )
    def gather_kernel(idx_hbm, table_hbm, out_hbm, idx_v, rows_v, sem):
        wid = lax.axis_index("s") * _NC + lax.axis_index("c")
        base = wid * b_per_w
        pltpu.sync_copy(idx_hbm.at[pl.ds(base, b_per_w)], idx_v)
        copies = []
        for c in range(n_chunks):
            copies.append(pltpu.async_copy(
                table_hbm.at[idx_v.at[pl.ds(c * _CHUNK, _CHUNK)]],
                rows_v.at[pl.ds(c * _CHUNK, _CHUNK)],
                sem))
        for cp in copies:
            cp.wait()
        pltpu.sync_copy(rows_v, out_hbm.at[pl.ds(base, b_per_w)])

    return gather_kernel


def kernel(h, W):
    N = h.shape[0] * h.shape[1]
    D = h.shape[2]
    K = W.shape[0]
    h_flat = h.reshape(N, D)

    idx, loss_sum = _tc_stage(h_flat, W, BN=512)
    q = _make_sc_gather(N, K, D)(idx, W)

    mse = loss_sum[0, 0] / jnp.float32(N * D)
    commitment_loss = jnp.float32(0.25) * mse
    codebook_loss = mse
    return q.reshape(h.shape), commitment_loss, codebook_loss
